# baseline (speedup 1.0000x reference)
import jax
import jax.numpy as jnp
from jax.experimental import pallas as pl
from jax.experimental.pallas import tpu as pltpu

_NCHUNK = 8


def _copy_body(src_ref, dst_ref, sems):
    rows = src_ref.shape[0]
    chunk = rows // _NCHUNK
    copies = [
        pltpu.make_async_copy(
            src_ref.at[pl.ds(i * chunk, chunk)],
            dst_ref.at[pl.ds(i * chunk, chunk)],
            sems.at[i],
        )
        for i in range(_NCHUNK)
    ]
    for c in copies:
        c.start()
    for c in copies:
        c.wait()


def kernel(x, emb_weight):
    del x
    out = pl.pallas_call(
        _copy_body,
        out_shape=jax.ShapeDtypeStruct(emb_weight.shape, emb_weight.dtype),
        in_specs=[pl.BlockSpec(memory_space=pl.MemorySpace.ANY)],
        out_specs=pl.BlockSpec(memory_space=pl.MemorySpace.ANY),
        scratch_shapes=[pltpu.SemaphoreType.DMA((_NCHUNK,))],
    )(emb_weight)
    return out[None]


# speedup vs baseline: 41.5125x; 41.5125x over previous
import jax
import jax.numpy as jnp
from jax.experimental import pallas as pl
from jax.experimental.pallas import tpu as pltpu

_BLOCK_ROWS = 512


def _copy_body(src_ref, dst_ref):
    dst_ref[...] = src_ref[...]


def kernel(x, emb_weight):
    del x
    rows, dim = emb_weight.shape
    out = pl.pallas_call(
        _copy_body,
        grid=(rows // _BLOCK_ROWS,),
        in_specs=[pl.BlockSpec((_BLOCK_ROWS, dim), lambda i: (i, 0))],
        out_specs=pl.BlockSpec((_BLOCK_ROWS, dim), lambda i: (i, 0)),
        out_shape=jax.ShapeDtypeStruct(emb_weight.shape, emb_weight.dtype),
    )(emb_weight)
    return out[None]
